# baseline (speedup 1.0000x reference)
import functools

import jax
import jax.numpy as jnp
from jax import lax
from jax.experimental import pallas as pl
from jax.experimental.pallas import tpu as pltpu
from jax.experimental.pallas import tpu_sc as plsc

_BATCH = 16384
_DIM = 64
_LANES = 16
_NW = 32
_NCOLS = (1000000 + 127) // 128
_CPW = (_NCOLS + _NW - 1) // _NW
_PAIR_CAP = _BATCH + 16
_NGRP = (_BATCH + 127) // 128
_ICHUNK = 2048


def _scan_sort(idx_hbm, idx_chunk_v, pairs_v, sorted_v, bidx_v,
               hist_s, cum_s, lo, nc, isem):
    iota = lax.iota(jnp.int32, _LANES)

    def init_b(r, _):
        for k in range(128 // _LANES):
            bidx_v[r, pl.ds(k * _LANES, _LANES)] = jnp.full(
                (_LANES,), _BATCH, jnp.int32)
        return 0
    lax.fori_loop(0, _NGRP + 1, init_b, 0)

    def init_h(c, _):
        hist_s[c] = 0
        return 0
    lax.fori_loop(0, 256, init_h, 0)

    nch = _BATCH // _ICHUNK
    pltpu.async_copy(idx_hbm.at[pl.ds(0, _ICHUNK)], idx_chunk_v.at[0], isem)

    def chunk_body(ch, cnt):
        @pl.when(ch + 1 < nch)
        def _():
            pltpu.async_copy(
                idx_hbm.at[pl.ds((ch + 1) * _ICHUNK, _ICHUNK)],
                idx_chunk_v.at[(ch + 1) & 1], isem)
        pltpu.make_async_copy(
            idx_hbm.at[pl.ds(0, _ICHUNK)], idx_chunk_v.at[ch & 1],
            isem).wait()

        def scan_body(g, cnt):
            u = idx_chunk_v[ch & 1, pl.ds(g * _LANES, _LANES)]
            col = u >> 7
            rel = col - lo
            m = (rel >= 0) & (rel < nc)
            b_vec = ch * _ICHUNK + g * _LANES + iota
            val = ((b_vec << 7) | (u & 127)) << 8 | rel
            plsc.store_compressed(pairs_v.at[pl.ds(cnt, _LANES)], val, mask=m)
            pc = plsc.all_reduce_population_count(m)
            return cnt + pc[0]

        return lax.fori_loop(0, _ICHUNK // _LANES, scan_body, cnt)

    cnt = lax.fori_loop(0, nch, chunk_body, 0)

    def hist_body(k, _):
        v = pairs_v[pl.ds(k * _LANES, _LANES)]
        for j in range(_LANES):
            @pl.when(k * _LANES + j < cnt)
            def _():
                c = v[j] & 255
                hist_s[c] = hist_s[c] + 1
        return 0
    lax.fori_loop(0, (cnt + _LANES - 1) // _LANES, hist_body, 0)

    def pfx_body(c, acc):
        cum_s[c] = acc
        n = hist_s[c]
        hist_s[c] = acc
        return acc + n
    total = lax.fori_loop(0, nc, pfx_body, 0)
    cum_s[nc] = total

    def place_body(k, _):
        v = pairs_v[pl.ds(k * _LANES, _LANES)]
        pos = jnp.zeros((_LANES,), jnp.int32)
        for j in range(_LANES):
            c = v[j] & 255
            p = hist_s[c]
            pos = jnp.where(iota == j, p, pos)
            @pl.when(k * _LANES + j < cnt)
            def _():
                hist_s[c] = p + 1
        valid = (k * _LANES + iota) < cnt
        data = v >> 8
        plsc.store_scatter(sorted_v, [pos], data, mask=valid)
        plsc.store_scatter(bidx_v, [pos >> 7, pos & 127], data >> 7,
                           mask=valid)
        return 0
    lax.fori_loop(0, (cnt + _LANES - 1) // _LANES, place_body, 0)
    return cnt


def _select_scalar(vec, j):
    iota = lax.iota(jnp.int32, _LANES)
    return jnp.sum(jnp.where(iota == j, vec, 0))


def _extract_row(colbuf2d, lane, dst2, r):
    iota = lax.iota(jnp.int32, _LANES)
    lane_vec = jnp.full((_LANES,), lane, jnp.int32)
    for k in range(_DIM // _LANES):
        dvec = k * _LANES + iota
        val = plsc.load_gather(colbuf2d, [dvec, lane_vec])
        dst2[r, pl.ds(k * _LANES, _LANES)] = val


def _col_dma(tabT_hbm, lo, cc, colbuf_v, slot, sem):
    return pltpu.async_copy(
        tabT_hbm.at[:, pl.ds((lo + cc) * 128, 128)], colbuf_v.at[slot], sem)


def _extract_kernel(user_hbm, item_hbm, utabT_hbm, itabT_hbm,
                    rows_u_hbm, rows_i_hbm,
                    idx_chunk_v, pairs_v, sorted_v, bidx_v, colbuf_v,
                    rows_seq_v, hist_s, cum_s, csem, wsem):
    wid = lax.axis_index("s") * 2 + lax.axis_index("c")
    lo = wid * _CPW
    nc = jnp.minimum(lo + _CPW, _NCOLS) - lo

    for idx_hbm, tabT_hbm, rows_hbm in (
            (user_hbm, utabT_hbm, rows_u_hbm),
            (item_hbm, itabT_hbm, rows_i_hbm)):
        cnt = _scan_sort(idx_hbm, idx_chunk_v, pairs_v, sorted_v, bidx_v,
                         hist_s, cum_s, lo, nc, csem)

        def nonempty(cc):
            return cum_s[cc] < cum_s[cc + 1]

        for p in range(6):
            @pl.when((p < nc) & nonempty(p))
            def _():
                _col_dma(tabT_hbm, lo, p, colbuf_v, p, csem)

        def col_body(cc, _):
            begin = cum_s[cc]
            end = cum_s[cc + 1]
            slot = lax.rem(cc, 7)

            @pl.when((cc + 6 < nc) & nonempty(cc + 6))
            def _prefetch():
                _col_dma(tabT_hbm, lo, cc + 6, colbuf_v, lax.rem(cc + 6, 7),
                         csem)

            @pl.when(begin < end)
            def _process():
                pltpu.make_async_copy(
                    tabT_hbm.at[:, pl.ds(lo * 128, 128)],
                    colbuf_v.at[slot], csem).wait()

                def pair_body(i, _):
                    v = sorted_v[pl.ds((i >> 4) << 4, _LANES)]
                    lb = _select_scalar(v, i & 15)
                    lane = lb & 127
                    r = i & 127
                    _extract_row(colbuf_v.at[slot], lane, rows_seq_v, r)

                    @pl.when((i & 127) == 127)
                    def _flush():
                        pltpu.async_copy(
                            rows_seq_v,
                            rows_hbm.at[bidx_v.at[i >> 7]], wsem).wait()
                    return 0

                lax.fori_loop(begin, end, pair_body, 0)
            return 0

        lax.fori_loop(0, nc, col_body, 0)

        @pl.when((cnt & 127) != 0)
        def _tail():
            pltpu.async_copy(
                rows_seq_v, rows_hbm.at[bidx_v.at[cnt >> 7]], wsem).wait()


def _dot_kernel(rows_u_hbm, rows_i_hbm, out_hbm,
                uch_v, ich_v, out_v, usem, isem):
    wid = lax.axis_index("s") * 2 + lax.axis_index("c")
    base = wid * (_BATCH // _NW)
    iota = lax.iota(jnp.int32, _LANES)

    def fire(c):
        return (
            pltpu.async_copy(
                rows_u_hbm.at[pl.ds(base + c * 128, 128)], uch_v.at[c & 1],
                usem),
            pltpu.async_copy(
                rows_i_hbm.at[pl.ds(base + c * 128, 128)], ich_v.at[c & 1],
                isem),
        )

    pending = fire(0)
    for c in range(4):
        nxt = fire(c + 1) if c + 1 < 4 else None
        pending[0].wait()
        pending[1].wait()

        def group_body(g, _, c=c):
            rvec = g * _LANES + iota
            acc = jnp.zeros((_LANES,), jnp.float32)
            for d in range(_DIM):
                dvec = jnp.full((_LANES,), d, jnp.int32)
                uu = plsc.load_gather(uch_v.at[c & 1], [rvec, dvec])
                vv = plsc.load_gather(ich_v.at[c & 1], [rvec, dvec])
                acc = acc + uu * vv
            out_v[pl.ds(c * 128 + g * _LANES, _LANES)] = acc
            return 0

        lax.fori_loop(0, 128 // _LANES, group_body, 0)
        pending = nxt

    pltpu.sync_copy(out_v, out_hbm.at[pl.ds(base, _BATCH // _NW)])


_COMPILER_PARAMS = pltpu.CompilerParams(
    use_tc_tiling_on_sc=True,
    needs_layout_passes=False,
)


@functools.partial(jax.jit, static_argnames=())
def kernel(user, item, user_table, item_table):
    mesh = plsc.VectorSubcoreMesh(core_axis_name="c", subcore_axis_name="s")

    run_extract = pl.kernel(
        _extract_kernel,
        mesh=mesh,
        compiler_params=_COMPILER_PARAMS,
        out_type=(jax.ShapeDtypeStruct((_BATCH + 1, 128), jnp.float32),
                  jax.ShapeDtypeStruct((_BATCH + 1, 128), jnp.float32)),
        scratch_types=[
            pltpu.VMEM((2, _ICHUNK), jnp.int32),
            pltpu.VMEM((_PAIR_CAP,), jnp.int32),
            pltpu.VMEM((_PAIR_CAP,), jnp.int32),
            pltpu.VMEM((_NGRP + 1, 128), jnp.int32),
            pltpu.VMEM((7, _DIM, 128), jnp.float32),
            pltpu.VMEM((128, 128), jnp.float32),
            pltpu.SMEM((256,), jnp.int32),
            pltpu.SMEM((256,), jnp.int32),
            pltpu.SemaphoreType.DMA,
            pltpu.SemaphoreType.DMA,
        ],
    )
    rows_u, rows_i = run_extract(user, item, user_table.T, item_table.T)

    run_dot = pl.kernel(
        _dot_kernel,
        mesh=mesh,
        compiler_params=_COMPILER_PARAMS,
        out_type=jax.ShapeDtypeStruct((_BATCH,), jnp.float32),
        scratch_types=[
            pltpu.VMEM((2, 128, 128), jnp.float32),
            pltpu.VMEM((2, 128, 128), jnp.float32),
            pltpu.VMEM((_BATCH // _NW,), jnp.float32),
            pltpu.SemaphoreType.DMA,
            pltpu.SemaphoreType.DMA,
        ],
    )
    return run_dot(rows_u, rows_i)


# speedup vs baseline: 1.1437x; 1.1437x over previous
import functools

import jax
import jax.numpy as jnp
from jax import lax
from jax.experimental import pallas as pl
from jax.experimental.pallas import tpu as pltpu
from jax.experimental.pallas import tpu_sc as plsc

_BATCH = 16384
_DIM = 64
_LANES = 16
_NW = 32
_NCOLS = (1000000 + 127) // 128
_CPW = (_NCOLS + _NW - 1) // _NW
_PAIR_CAP = _BATCH + 16
_NGRP = (_BATCH + 127) // 128
_ICHUNK = 2048


def _scan_sort(idx_hbm, idx_chunk_v, pairs_v, sorted_v, bidx_v,
               hist_s, cum_s, lo, nc, isem):
    iota = lax.iota(jnp.int32, _LANES)

    def init_b(r, _):
        for k in range(128 // _LANES):
            bidx_v[r, pl.ds(k * _LANES, _LANES)] = jnp.full(
                (_LANES,), _BATCH, jnp.int32)
        return 0
    lax.fori_loop(0, _NGRP + 1, init_b, 0)

    def init_h(c, _):
        hist_s[c] = 0
        return 0
    lax.fori_loop(0, 256, init_h, 0)

    nch = _BATCH // _ICHUNK
    pltpu.async_copy(idx_hbm.at[pl.ds(0, _ICHUNK)], idx_chunk_v.at[0], isem)

    def chunk_body(ch, cnt):
        @pl.when(ch + 1 < nch)
        def _():
            pltpu.async_copy(
                idx_hbm.at[pl.ds((ch + 1) * _ICHUNK, _ICHUNK)],
                idx_chunk_v.at[(ch + 1) & 1], isem)
        pltpu.make_async_copy(
            idx_hbm.at[pl.ds(0, _ICHUNK)], idx_chunk_v.at[ch & 1],
            isem).wait()

        def scan_body(g, cnt):
            u = idx_chunk_v[ch & 1, pl.ds(g * _LANES, _LANES)]
            col = u >> 7
            rel = col - lo
            m = (rel >= 0) & (rel < nc)
            b_vec = ch * _ICHUNK + g * _LANES + iota
            val = ((b_vec << 7) | (u & 127)) << 8 | rel
            plsc.store_compressed(pairs_v.at[pl.ds(cnt, _LANES)], val, mask=m)
            pc = plsc.all_reduce_population_count(m)
            return cnt + pc[0]

        return lax.fori_loop(0, _ICHUNK // _LANES, scan_body, cnt)

    cnt = lax.fori_loop(0, nch, chunk_body, 0)

    def hist_body(k, _):
        v = pairs_v[pl.ds(k * _LANES, _LANES)]
        for j in range(_LANES):
            @pl.when(k * _LANES + j < cnt)
            def _():
                c = v[j] & 255
                hist_s[c] = hist_s[c] + 1
        return 0
    lax.fori_loop(0, (cnt + _LANES - 1) // _LANES, hist_body, 0)

    def pfx_body(c, acc):
        cum_s[c] = acc
        n = hist_s[c]
        hist_s[c] = acc
        return acc + n
    total = lax.fori_loop(0, nc, pfx_body, 0)
    cum_s[nc] = total

    def place_body(k, _):
        v = pairs_v[pl.ds(k * _LANES, _LANES)]
        pos = jnp.zeros((_LANES,), jnp.int32)
        for j in range(_LANES):
            c = v[j] & 255
            p = hist_s[c]
            pos = jnp.where(iota == j, p, pos)
            @pl.when(k * _LANES + j < cnt)
            def _():
                hist_s[c] = p + 1
        valid = (k * _LANES + iota) < cnt
        data = v >> 8
        plsc.store_scatter(sorted_v, [pos], data, mask=valid)
        plsc.store_scatter(bidx_v, [pos >> 7, pos & 127], data >> 7,
                           mask=valid)
        return 0
    lax.fori_loop(0, (cnt + _LANES - 1) // _LANES, place_body, 0)
    return cnt


def _select_scalar(vec, j):
    iota = lax.iota(jnp.int32, _LANES)
    return jnp.sum(jnp.where(iota == j, vec, 0))


def _extract_row(colbuf2d, lane, dst2, r):
    iota = lax.iota(jnp.int32, _LANES)
    lane_vec = jnp.full((_LANES,), lane, jnp.int32)
    for k in range(_DIM // _LANES):
        dvec = k * _LANES + iota
        val = plsc.load_gather(colbuf2d, [dvec, lane_vec])
        dst2[r, pl.ds(k * _LANES, _LANES)] = val


def _col_dma(tabT_hbm, lo, cc, colbuf_v, slot, sem):
    return pltpu.async_copy(
        tabT_hbm.at[:, pl.ds((lo + cc) * 128, 128)], colbuf_v.at[slot], sem)


def _users_kernel(user_hbm, utabT_hbm, rows_u_hbm,
                  idx_chunk_v, pairs_v, sorted_v, bidx_v, colbuf_v,
                  rows_seq_v, hist_s, cum_s, csem, wsem):
    wid = lax.axis_index("s") * 2 + lax.axis_index("c")
    lo = wid * _CPW
    nc = jnp.minimum(lo + _CPW, _NCOLS) - lo

    cnt = _scan_sort(user_hbm, idx_chunk_v, pairs_v, sorted_v, bidx_v,
                     hist_s, cum_s, lo, nc, csem)

    def nonempty(cc):
        return cum_s[cc] < cum_s[cc + 1]

    for p in range(6):
        @pl.when((p < nc) & nonempty(p))
        def _():
            _col_dma(utabT_hbm, lo, p, colbuf_v, p, csem)

    def col_body(cc, _):
        begin = cum_s[cc]
        end = cum_s[cc + 1]
        slot = lax.rem(cc, 7)

        @pl.when((cc + 6 < nc) & nonempty(cc + 6))
        def _prefetch():
            _col_dma(utabT_hbm, lo, cc + 6, colbuf_v, lax.rem(cc + 6, 7),
                     csem)

        @pl.when(begin < end)
        def _process():
            pltpu.make_async_copy(
                utabT_hbm.at[:, pl.ds(lo * 128, 128)],
                colbuf_v.at[slot], csem).wait()

            def pair_body(i, _):
                v = sorted_v[pl.ds((i >> 4) << 4, _LANES)]
                lb = _select_scalar(v, i & 15)
                lane = lb & 127
                r = i & 127
                _extract_row(colbuf_v.at[slot], lane, rows_seq_v, r)

                @pl.when((i & 127) == 127)
                def _flush():
                    pltpu.async_copy(
                        rows_seq_v,
                        rows_u_hbm.at[bidx_v.at[i >> 7]], wsem).wait()
                return 0

            lax.fori_loop(begin, end, pair_body, 0)
        return 0

    lax.fori_loop(0, nc, col_body, 0)

    @pl.when((cnt & 127) != 0)
    def _tail():
        pltpu.async_copy(
            rows_seq_v, rows_u_hbm.at[bidx_v.at[cnt >> 7]], wsem).wait()


def _dot_kernel(rows_u_hbm, rows_i_hbm, out_hbm,
                uch_v, ich_v, out_v, usem, isem):
    wid = lax.axis_index("s") * 2 + lax.axis_index("c")
    base = wid * (_BATCH // _NW)
    iota = lax.iota(jnp.int32, _LANES)

    def fire(c):
        return (
            pltpu.async_copy(
                rows_u_hbm.at[pl.ds(base + c * 128, 128)], uch_v.at[c & 1],
                usem),
            pltpu.async_copy(
                rows_i_hbm.at[pl.ds(base + c * 128, 128)], ich_v.at[c & 1],
                isem),
        )

    pending = fire(0)
    for c in range(4):
        nxt = fire(c + 1) if c + 1 < 4 else None
        pending[0].wait()
        pending[1].wait()

        def group_body(g, _, c=c):
            rvec = g * _LANES + iota
            acc = jnp.zeros((_LANES,), jnp.float32)
            for d in range(_DIM):
                dvec = jnp.full((_LANES,), d, jnp.int32)
                uu = plsc.load_gather(uch_v.at[c & 1], [rvec, dvec])
                vv = plsc.load_gather(ich_v.at[c & 1], [rvec, dvec])
                acc = acc + uu * vv
            out_v[pl.ds(c * 128 + g * _LANES, _LANES)] = acc
            return 0

        lax.fori_loop(0, 128 // _LANES, group_body, 0)
        pending = nxt

    pltpu.sync_copy(out_v, out_hbm.at[pl.ds(base, _BATCH // _NW)])


_COMPILER_PARAMS = pltpu.CompilerParams(
    use_tc_tiling_on_sc=True,
    needs_layout_passes=False,
)


@functools.partial(jax.jit, static_argnames=())
def kernel(user, item, user_table, item_table):
    mesh = plsc.VectorSubcoreMesh(core_axis_name="c", subcore_axis_name="s")

    run_extract = pl.kernel(
        _users_kernel,
        mesh=mesh,
        compiler_params=_COMPILER_PARAMS,
        out_type=jax.ShapeDtypeStruct((_BATCH + 1, 128), jnp.float32),
        scratch_types=[
            pltpu.VMEM((2, _ICHUNK), jnp.int32),
            pltpu.VMEM((_PAIR_CAP,), jnp.int32),
            pltpu.VMEM((_PAIR_CAP,), jnp.int32),
            pltpu.VMEM((_NGRP + 1, 128), jnp.int32),
            pltpu.VMEM((7, _DIM, 128), jnp.float32),
            pltpu.VMEM((128, 128), jnp.float32),
            pltpu.SMEM((256,), jnp.int32),
            pltpu.SMEM((256,), jnp.int32),
            pltpu.SemaphoreType.DMA,
            pltpu.SemaphoreType.DMA,
        ],
    )
    rows_u = run_extract(user, user_table.T)
    rows_i = run_extract(item, item_table.T)

    run_dot = pl.kernel(
        _dot_kernel,
        mesh=mesh,
        compiler_params=_COMPILER_PARAMS,
        out_type=jax.ShapeDtypeStruct((_BATCH,), jnp.float32),
        scratch_types=[
            pltpu.VMEM((2, 128, 128), jnp.float32),
            pltpu.VMEM((2, 128, 128), jnp.float32),
            pltpu.VMEM((_BATCH // _NW,), jnp.float32),
            pltpu.SemaphoreType.DMA,
            pltpu.SemaphoreType.DMA,
        ],
    )
    return run_dot(rows_u, rows_i)
